# baseline (speedup 1.0000x reference)
import functools

import jax
import jax.numpy as jnp
from jax import lax
from jax.experimental import pallas as pl
from jax.experimental.pallas import tpu as pltpu
from jax.experimental.pallas import tpu_sc as plsc

E = 64
TOPK = 2
H = 768
F = 1024
N = 4096
A = N * TOPK
T = 128
P = A + E * T
NT = P // T

NC = 2
NS = 16
NW = NC * NS

_SQRT_HALF = 0.7071067811865476


def _gelu_exact(v):
    return 0.5 * v * (1.0 + lax.erf(v * _SQRT_HALF))


_RB = 512


def _router_body(x_ref, gw_ref, gb_ref, idx_ref, rw_ref, rank_ref, g_ref,
                 cnt_ref):
    blk = pl.program_id(0)
    x = x_ref[...]
    logits = lax.dot_general(x, gw_ref[...], (((1,), (1,)), ((), ())),
                             preferred_element_type=jnp.float32)
    logits = logits + gb_ref[...][None, :]
    ids = lax.broadcasted_iota(jnp.int32, logits.shape, 1)
    neg = jnp.float32(jnp.finfo(jnp.float32).min)
    m1 = jnp.max(logits, axis=1, keepdims=True)
    a1 = jnp.min(jnp.where(logits == m1, ids, E), axis=1, keepdims=True)
    l2 = jnp.where(ids == a1, neg, logits)
    m2 = jnp.max(l2, axis=1, keepdims=True)
    a2 = jnp.min(jnp.where(l2 == m2, ids, E), axis=1, keepdims=True)
    t = jnp.exp(m2 - m1)
    w1 = 1.0 / (1.0 + t)
    idx_ref[...] = jnp.concatenate([a1, a2], axis=1)
    rw_ref[...] = jnp.concatenate([w1, t * w1], axis=1)

    @pl.when(blk == 0)
    def _():
        cnt_ref[...] = jnp.zeros((1, E), jnp.float32)

    oh0 = (ids == a1).astype(jnp.float32)
    oh1 = (ids == a2).astype(jnp.float32)
    s = oh0 + oh1
    incl = s
    k = 1
    while k < _RB:
        shifted = jnp.concatenate(
            [jnp.zeros((k, E), jnp.float32), incl[:-k, :]], axis=0)
        incl = incl + shifted
        k *= 2
    excl = incl - s
    base = cnt_ref[...] + excl
    r0 = jnp.sum(oh0 * base, axis=1, keepdims=True)
    r1 = jnp.sum(oh1 * base, axis=1, keepdims=True)
    rank_ref[...] = jnp.concatenate([r0, r1], axis=1).astype(jnp.int32)
    cnt_new = cnt_ref[...] + incl[-1:, :]
    cnt_ref[...] = cnt_new
    g_ref[...] = cnt_new.astype(jnp.int32)


def _router(xf, gate_w, gate_b):
    return pl.pallas_call(
        _router_body,
        grid=(N // _RB,),
        in_specs=[
            pl.BlockSpec((_RB, H), lambda i: (i, 0)),
            pl.BlockSpec((E, H), lambda i: (0, 0)),
            pl.BlockSpec((E,), lambda i: (0,)),
        ],
        out_specs=[
            pl.BlockSpec((_RB, TOPK), lambda i: (i, 0)),
            pl.BlockSpec((_RB, TOPK), lambda i: (i, 0)),
            pl.BlockSpec((_RB, TOPK), lambda i: (i, 0)),
            pl.BlockSpec((1, E), lambda i: (0, 0)),
        ],
        out_shape=[
            jax.ShapeDtypeStruct((N, TOPK), jnp.int32),
            jax.ShapeDtypeStruct((N, TOPK), jnp.float32),
            jax.ShapeDtypeStruct((N, TOPK), jnp.int32),
            jax.ShapeDtypeStruct((1, E), jnp.int32),
        ],
        scratch_shapes=[pltpu.VMEM((1, E), jnp.float32)],
        compiler_params=pltpu.CompilerParams(dimension_semantics=("arbitrary",)),
    )(xf, gate_w, gate_b)


_DTW = N // NW
_DHF = _DTW // 2


@functools.lru_cache(maxsize=None)
def _make_sc_dispatch():
    mesh = plsc.VectorSubcoreMesh(
        core_axis_name="c", subcore_axis_name="s",
        num_cores=NC, num_subcores=NS)

    @functools.partial(
        pl.kernel,
        out_type=jax.ShapeDtypeStruct((P, H), jnp.float32),
        mesh=mesh,
        scratch_types=[
            pltpu.VMEM((_DTW, H), jnp.float32),
            [pltpu.VMEM((_DHF,), jnp.int32) for _ in range(4)],
            pltpu.SemaphoreType.DMA,
        ],
    )
    def dispatch_k(sk0_hbm, sk1_hbm, x_hbm, out_hbm, buf, idxs, sem):
        wid = lax.axis_index("s") * NC + lax.axis_index("c")
        t0 = wid * _DTW
        pltpu.sync_copy(x_hbm.at[pl.ds(t0, _DTW)], buf)
        pltpu.sync_copy(sk0_hbm.at[pl.ds(t0, _DHF)], idxs[0])
        pltpu.sync_copy(sk0_hbm.at[pl.ds(t0 + _DHF, _DHF)], idxs[1])
        pltpu.sync_copy(sk1_hbm.at[pl.ds(t0, _DHF)], idxs[2])
        pltpu.sync_copy(sk1_hbm.at[pl.ds(t0 + _DHF, _DHF)], idxs[3])
        handles = [
            pltpu.async_copy(buf.at[pl.ds(0, _DHF)], out_hbm.at[idxs[0]], sem),
            pltpu.async_copy(buf.at[pl.ds(_DHF, _DHF)], out_hbm.at[idxs[1]], sem),
            pltpu.async_copy(buf.at[pl.ds(0, _DHF)], out_hbm.at[idxs[2]], sem),
            pltpu.async_copy(buf.at[pl.ds(_DHF, _DHF)], out_hbm.at[idxs[3]], sem),
        ]
        for h in handles:
            h.wait()

    return dispatch_k


def _sc_dispatch(slots_k0, slots_k1, xf):
    return _make_sc_dispatch()(slots_k0, slots_k1, xf)


def _ffn_body(er_ref, nr_ref, x_ref, w1_ref, b1_ref, w2_ref, b2_ref, o_ref):
    t = pl.program_id(0)

    @pl.when(t < nr_ref[0])
    def _():
        x = x_ref[...]
        h = lax.dot_general(x, w1_ref[0], (((1,), (1,)), ((), ())),
                            preferred_element_type=jnp.float32)
        h = _gelu_exact(h + b1_ref[0])
        o = lax.dot_general(h, w2_ref[0], (((1,), (1,)), ((), ())),
                            preferred_element_type=jnp.float32)
        o_ref[...] = o + b2_ref[0]


def _gmm(expert_of_tile, n_active, x_sorted, fc1_w, fc1_b, fc2_w, fc2_b):
    def rowblk(t, er, nr):
        return (jnp.minimum(t, nr[0] - 1), 0)

    grid_spec = pltpu.PrefetchScalarGridSpec(
        num_scalar_prefetch=2,
        grid=(NT,),
        in_specs=[
            pl.BlockSpec((T, H), rowblk),
            pl.BlockSpec((1, F, H), lambda t, er, nr: (er[t], 0, 0)),
            pl.BlockSpec((1, 1, F), lambda t, er, nr: (er[t], 0, 0)),
            pl.BlockSpec((1, H, F), lambda t, er, nr: (er[t], 0, 0)),
            pl.BlockSpec((1, 1, H), lambda t, er, nr: (er[t], 0, 0)),
        ],
        out_specs=pl.BlockSpec((T, H), rowblk),
    )
    return pl.pallas_call(
        _ffn_body,
        grid_spec=grid_spec,
        out_shape=jax.ShapeDtypeStruct((P, H), jnp.float32),
        compiler_params=pltpu.CompilerParams(dimension_semantics=("arbitrary",)),
    )(expert_of_tile, n_active, x_sorted, fc1_w, fc1_b.reshape(E, 1, F),
      fc2_w, fc2_b.reshape(E, 1, H))


_CT = 8
_CDEPTH = 4


@functools.lru_cache(maxsize=None)
def _make_sc_combine():
    mesh = plsc.VectorSubcoreMesh(
        core_axis_name="c", subcore_axis_name="s",
        num_cores=NC, num_subcores=NS)
    n_tok = N // NW
    nch = n_tok // _CT

    @functools.partial(
        pl.kernel,
        out_type=jax.ShapeDtypeStruct((N, H), jnp.float32),
        mesh=mesh,
        scratch_types=[
            pltpu.VMEM((TOPK * n_tok,), jnp.int32),
            [pltpu.VMEM((2 * _CT, H), jnp.float32) for _ in range(_CDEPTH)],
            [pltpu.VMEM((_CT, H), jnp.float32) for _ in range(_CDEPTH)],
            pltpu.VMEM((TOPK * n_tok,), jnp.float32),
            [pltpu.SemaphoreType.DMA for _ in range(_CDEPTH)],
            [pltpu.SemaphoreType.DMA for _ in range(_CDEPTH)],
        ],
        compiler_params=pltpu.CompilerParams(needs_layout_passes=False),
    )
    def combine_k(slots_hbm, rw_hbm, o_hbm, y_hbm, idx_all, rows_v, y_v, w_v,
                  gsem, wsem):
        wid = lax.axis_index("s") * NC + lax.axis_index("c")
        base_t = wid * n_tok
        pltpu.sync_copy(slots_hbm.at[pl.ds(TOPK * base_t, TOPK * n_tok)],
                        idx_all)
        pltpu.sync_copy(rw_hbm.at[pl.ds(TOPK * base_t, TOPK * n_tok)], w_v)

        def start_gather(j, b):
            idx = idx_all.at[pl.ds(TOPK * _CT * j, TOPK * _CT)]
            return pltpu.async_copy(o_hbm.at[idx], rows_v[b], gsem[b])

        gh = [start_gather(j, j) for j in range(_CDEPTH)]
        wh = [None] * _CDEPTH
        for j in range(nch):
            b = j % _CDEPTH
            gh[b].wait()
            if wh[b] is not None:
                wh[b].wait()

            def per_tok(i, c1, j=j, b=b):
                iw = TOPK * (j * _CT + i)
                wa = plsc.load_gather(w_v, [jnp.full((16,), iw, jnp.int32)])
                wb = plsc.load_gather(w_v, [jnp.full((16,), iw + 1,
                                                     jnp.int32)])
                for c in range(H // 16):
                    a = rows_v[b][2 * i, pl.ds(c * 16, 16)]
                    bb = rows_v[b][2 * i + 1, pl.ds(c * 16, 16)]
                    y_v[b][i, pl.ds(c * 16, 16)] = wa * a + wb * bb
                return c1

            lax.fori_loop(0, _CT, per_tok, 0)
            wh[b] = pltpu.async_copy(
                y_v[b], y_hbm.at[pl.ds(base_t + j * _CT, _CT)], wsem[b])
            if j + _CDEPTH < nch:
                gh[b] = start_gather(j + _CDEPTH, b)
        for h in wh:
            if h is not None:
                h.wait()

    return combine_k


def _sc_combine(slot_of_assign, rw_flat, o_out):
    return _make_sc_combine()(slot_of_assign, rw_flat, o_out)


def _dispatch_indices(top_idx, rank, g_row):
    g = g_row[0]
    gpad = ((g + T - 1) // T) * T
    padded_end = jnp.cumsum(gpad).astype(jnp.int32)
    padded_start = padded_end - gpad
    slot_of_assign = (padded_start[top_idx] + rank).reshape(-1)
    tile_rows = jnp.arange(NT, dtype=jnp.int32) * T
    expert_of_tile = jnp.minimum(
        jnp.searchsorted(padded_end, tile_rows, side="right"), E - 1
    ).astype(jnp.int32)
    n_active = (padded_end[-1] // T).astype(jnp.int32).reshape(1)
    return slot_of_assign, expert_of_tile, n_active


def kernel(x, gate_w, gate_b, fc1_w, fc1_b, fc2_w, fc2_b):
    Bs, Ss, Hd = x.shape
    xf = x.reshape(-1, Hd)
    top_idx, rw, rank, g_row = _router(xf, gate_w, gate_b)
    slot_of_assign, expert_of_tile, n_active = _dispatch_indices(
        top_idx, rank, g_row)
    slot_tk = slot_of_assign.reshape(N, TOPK)
    x_sorted = _sc_dispatch(slot_tk[:, 0], slot_tk[:, 1], xf)
    o_out = _gmm(expert_of_tile, n_active, x_sorted, fc1_w, fc1_b,
                 fc2_w, fc2_b)
    y = _sc_combine(slot_of_assign, rw.reshape(-1), o_out)
    return y.reshape(Bs, Ss, Hd)


# speedup vs baseline: 1.0676x; 1.0676x over previous
import functools

import jax
import jax.numpy as jnp
from jax import lax
from jax.experimental import pallas as pl
from jax.experimental.pallas import tpu as pltpu
from jax.experimental.pallas import tpu_sc as plsc

E = 64
TOPK = 2
H = 768
F = 1024
N = 4096
A = N * TOPK
T = 128
P = A + E * T
NT = P // T

NC = 2
NS = 16
NW = NC * NS

_SQRT_HALF = 0.7071067811865476


def _gelu_exact(v):
    return 0.5 * v * (1.0 + lax.erf(v * _SQRT_HALF))


_RB = 512


def _router_body(x_ref, gw_ref, gb_ref, idx_ref, rw_ref, rank_ref, g_ref,
                 cnt_ref):
    blk = pl.program_id(0)
    x = x_ref[...]
    logits = lax.dot_general(x, gw_ref[...], (((1,), (1,)), ((), ())),
                             preferred_element_type=jnp.float32)
    logits = logits + gb_ref[...][None, :]
    ids = lax.broadcasted_iota(jnp.int32, logits.shape, 1)
    neg = jnp.float32(jnp.finfo(jnp.float32).min)
    m1 = jnp.max(logits, axis=1, keepdims=True)
    a1 = jnp.min(jnp.where(logits == m1, ids, E), axis=1, keepdims=True)
    l2 = jnp.where(ids == a1, neg, logits)
    m2 = jnp.max(l2, axis=1, keepdims=True)
    a2 = jnp.min(jnp.where(l2 == m2, ids, E), axis=1, keepdims=True)
    t = jnp.exp(m2 - m1)
    w1 = 1.0 / (1.0 + t)
    idx_ref[...] = jnp.concatenate([a1, a2], axis=1)
    rw_ref[...] = jnp.concatenate([w1, t * w1], axis=1)

    @pl.when(blk == 0)
    def _():
        cnt_ref[...] = jnp.zeros((1, E), jnp.float32)

    oh0 = (ids == a1).astype(jnp.float32)
    oh1 = (ids == a2).astype(jnp.float32)
    s = oh0 + oh1
    incl = s
    k = 1
    while k < _RB:
        shifted = jnp.concatenate(
            [jnp.zeros((k, E), jnp.float32), incl[:-k, :]], axis=0)
        incl = incl + shifted
        k *= 2
    excl = incl - s
    base = cnt_ref[...] + excl
    r0 = jnp.sum(oh0 * base, axis=1, keepdims=True)
    r1 = jnp.sum(oh1 * base, axis=1, keepdims=True)
    rank_ref[...] = jnp.concatenate([r0, r1], axis=1).astype(jnp.int32)
    cnt_new = cnt_ref[...] + incl[-1:, :]
    cnt_ref[...] = cnt_new
    g_ref[...] = cnt_new.astype(jnp.int32)


def _router(xf, gate_w, gate_b):
    return pl.pallas_call(
        _router_body,
        grid=(N // _RB,),
        in_specs=[
            pl.BlockSpec((_RB, H), lambda i: (i, 0)),
            pl.BlockSpec((E, H), lambda i: (0, 0)),
            pl.BlockSpec((E,), lambda i: (0,)),
        ],
        out_specs=[
            pl.BlockSpec((_RB, TOPK), lambda i: (i, 0)),
            pl.BlockSpec((_RB, TOPK), lambda i: (i, 0)),
            pl.BlockSpec((_RB, TOPK), lambda i: (i, 0)),
            pl.BlockSpec((1, E), lambda i: (0, 0)),
        ],
        out_shape=[
            jax.ShapeDtypeStruct((N, TOPK), jnp.int32),
            jax.ShapeDtypeStruct((N, TOPK), jnp.float32),
            jax.ShapeDtypeStruct((N, TOPK), jnp.int32),
            jax.ShapeDtypeStruct((1, E), jnp.int32),
        ],
        scratch_shapes=[pltpu.VMEM((1, E), jnp.float32)],
        compiler_params=pltpu.CompilerParams(dimension_semantics=("arbitrary",)),
    )(xf, gate_w, gate_b)


_DTW = N // NW
_DHF = _DTW // 2


@functools.lru_cache(maxsize=None)
def _make_sc_dispatch():
    mesh = plsc.VectorSubcoreMesh(
        core_axis_name="c", subcore_axis_name="s",
        num_cores=NC, num_subcores=NS)

    @functools.partial(
        pl.kernel,
        out_type=jax.ShapeDtypeStruct((P, H), jnp.float32),
        mesh=mesh,
        scratch_types=[
            pltpu.VMEM((_DTW, H), jnp.float32),
            [pltpu.VMEM((_DHF,), jnp.int32) for _ in range(4)],
            pltpu.SemaphoreType.DMA,
        ],
    )
    def dispatch_k(sk0_hbm, sk1_hbm, x_hbm, out_hbm, buf, idxs, sem):
        wid = lax.axis_index("s") * NC + lax.axis_index("c")
        t0 = wid * _DTW
        pltpu.sync_copy(x_hbm.at[pl.ds(t0, _DTW)], buf)
        pltpu.sync_copy(sk0_hbm.at[pl.ds(t0, _DHF)], idxs[0])
        pltpu.sync_copy(sk0_hbm.at[pl.ds(t0 + _DHF, _DHF)], idxs[1])
        pltpu.sync_copy(sk1_hbm.at[pl.ds(t0, _DHF)], idxs[2])
        pltpu.sync_copy(sk1_hbm.at[pl.ds(t0 + _DHF, _DHF)], idxs[3])
        handles = [
            pltpu.async_copy(buf.at[pl.ds(0, _DHF)], out_hbm.at[idxs[0]], sem),
            pltpu.async_copy(buf.at[pl.ds(_DHF, _DHF)], out_hbm.at[idxs[1]], sem),
            pltpu.async_copy(buf.at[pl.ds(0, _DHF)], out_hbm.at[idxs[2]], sem),
            pltpu.async_copy(buf.at[pl.ds(_DHF, _DHF)], out_hbm.at[idxs[3]], sem),
        ]
        for h in handles:
            h.wait()

    return dispatch_k


def _sc_dispatch(slots_k0, slots_k1, xf):
    return _make_sc_dispatch()(slots_k0, slots_k1, xf)


def _ffn_body(er_ref, nr_ref, x_ref, w1_ref, b1_ref, w2_ref, b2_ref, o_ref):
    t = pl.program_id(0)

    @pl.when(t < nr_ref[0])
    def _():
        x = x_ref[...]
        h = lax.dot_general(x, w1_ref[0], (((1,), (1,)), ((), ())),
                            preferred_element_type=jnp.float32)
        h = _gelu_exact(h + b1_ref[0])
        o = lax.dot_general(h, w2_ref[0], (((1,), (1,)), ((), ())),
                            preferred_element_type=jnp.float32)
        o = o + b2_ref[0]
        oi = lax.bitcast_convert_type(o, jnp.int32) + jnp.int32(0x8000)
        lo = jnp.right_shift(oi[:, :H // 2], 16) & jnp.int32(0xFFFF)
        hi = oi[:, H // 2:] & jnp.int32(-65536)
        o_ref[...] = hi | lo


def _gmm(expert_of_tile, n_active, x_sorted, fc1_w, fc1_b, fc2_w, fc2_b):
    def rowblk(t, er, nr):
        return (jnp.minimum(t, nr[0] - 1), 0)

    grid_spec = pltpu.PrefetchScalarGridSpec(
        num_scalar_prefetch=2,
        grid=(NT,),
        in_specs=[
            pl.BlockSpec((T, H), rowblk),
            pl.BlockSpec((1, F, H), lambda t, er, nr: (er[t], 0, 0)),
            pl.BlockSpec((1, 1, F), lambda t, er, nr: (er[t], 0, 0)),
            pl.BlockSpec((1, H, F), lambda t, er, nr: (er[t], 0, 0)),
            pl.BlockSpec((1, 1, H), lambda t, er, nr: (er[t], 0, 0)),
        ],
        out_specs=pl.BlockSpec((T, H // 2), rowblk),
    )
    return pl.pallas_call(
        _ffn_body,
        grid_spec=grid_spec,
        out_shape=jax.ShapeDtypeStruct((P, H // 2), jnp.int32),
        compiler_params=pltpu.CompilerParams(dimension_semantics=("arbitrary",)),
    )(expert_of_tile, n_active, x_sorted, fc1_w, fc1_b.reshape(E, 1, F),
      fc2_w, fc2_b.reshape(E, 1, H))


_CT = 8
_CDEPTH = 4


@functools.lru_cache(maxsize=None)
def _make_sc_combine():
    mesh = plsc.VectorSubcoreMesh(
        core_axis_name="c", subcore_axis_name="s",
        num_cores=NC, num_subcores=NS)
    n_tok = N // NW
    nch = n_tok // _CT

    @functools.partial(
        pl.kernel,
        out_type=jax.ShapeDtypeStruct((N, H), jnp.float32),
        mesh=mesh,
        scratch_types=[
            pltpu.VMEM((TOPK * n_tok,), jnp.int32),
            [pltpu.VMEM((2 * _CT, H // 2), jnp.int32) for _ in range(_CDEPTH)],
            [pltpu.VMEM((_CT, H), jnp.float32) for _ in range(_CDEPTH)],
            pltpu.VMEM((TOPK * n_tok,), jnp.float32),
            [pltpu.SemaphoreType.DMA for _ in range(_CDEPTH)],
            [pltpu.SemaphoreType.DMA for _ in range(_CDEPTH)],
        ],
        compiler_params=pltpu.CompilerParams(needs_layout_passes=False),
    )
    def combine_k(slots_hbm, rw_hbm, o_hbm, y_hbm, idx_all, rows_v, y_v, w_v,
                  gsem, wsem):
        wid = lax.axis_index("s") * NC + lax.axis_index("c")
        base_t = wid * n_tok
        pltpu.sync_copy(slots_hbm.at[pl.ds(TOPK * base_t, TOPK * n_tok)],
                        idx_all)
        pltpu.sync_copy(rw_hbm.at[pl.ds(TOPK * base_t, TOPK * n_tok)], w_v)

        def start_gather(j, b):
            idx = idx_all.at[pl.ds(TOPK * _CT * j, TOPK * _CT)]
            return pltpu.async_copy(o_hbm.at[idx], rows_v[b], gsem[b])

        gh = [start_gather(j, j) for j in range(_CDEPTH)]
        wh = [None] * _CDEPTH
        for j in range(nch):
            b = j % _CDEPTH
            gh[b].wait()
            if wh[b] is not None:
                wh[b].wait()

            def per_tok(i, c1, j=j, b=b):
                iw = TOPK * (j * _CT + i)
                wa = plsc.load_gather(w_v, [jnp.full((16,), iw, jnp.int32)])
                wb = plsc.load_gather(w_v, [jnp.full((16,), iw + 1,
                                                     jnp.int32)])
                hmask = jnp.full((16,), -65536, jnp.int32)
                for c in range(H // 32):
                    aw = rows_v[b][2 * i, pl.ds(c * 16, 16)]
                    bw = rows_v[b][2 * i + 1, pl.ds(c * 16, 16)]
                    a_lo = plsc.bitcast(jnp.left_shift(aw, 16), jnp.float32)
                    b_lo = plsc.bitcast(jnp.left_shift(bw, 16), jnp.float32)
                    a_hi = plsc.bitcast(aw & hmask, jnp.float32)
                    b_hi = plsc.bitcast(bw & hmask, jnp.float32)
                    y_v[b][i, pl.ds(c * 16, 16)] = wa * a_lo + wb * b_lo
                    y_v[b][i, pl.ds(H // 2 + c * 16, 16)] = (
                        wa * a_hi + wb * b_hi)
                return c1

            lax.fori_loop(0, _CT, per_tok, 0)
            wh[b] = pltpu.async_copy(
                y_v[b], y_hbm.at[pl.ds(base_t + j * _CT, _CT)], wsem[b])
            if j + _CDEPTH < nch:
                gh[b] = start_gather(j + _CDEPTH, b)
        for h in wh:
            if h is not None:
                h.wait()

    return combine_k


def _sc_combine(slot_of_assign, rw_flat, o_out):
    return _make_sc_combine()(slot_of_assign, rw_flat, o_out)


def _dispatch_indices(top_idx, rank, g_row):
    g = g_row[0]
    gpad = ((g + T - 1) // T) * T
    padded_end = jnp.cumsum(gpad).astype(jnp.int32)
    padded_start = padded_end - gpad
    slot_of_assign = (padded_start[top_idx] + rank).reshape(-1)
    tile_rows = jnp.arange(NT, dtype=jnp.int32) * T
    expert_of_tile = jnp.minimum(
        jnp.searchsorted(padded_end, tile_rows, side="right"), E - 1
    ).astype(jnp.int32)
    n_active = (padded_end[-1] // T).astype(jnp.int32).reshape(1)
    return slot_of_assign, expert_of_tile, n_active


def kernel(x, gate_w, gate_b, fc1_w, fc1_b, fc2_w, fc2_b):
    Bs, Ss, Hd = x.shape
    xf = x.reshape(-1, Hd)
    top_idx, rw, rank, g_row = _router(xf, gate_w, gate_b)
    slot_of_assign, expert_of_tile, n_active = _dispatch_indices(
        top_idx, rank, g_row)
    slot_tk = slot_of_assign.reshape(N, TOPK)
    x_sorted = _sc_dispatch(slot_tk[:, 0], slot_tk[:, 1], xf)
    o_out = _gmm(expert_of_tile, n_active, x_sorted, fc1_w, fc1_b,
                 fc2_w, fc2_b)
    y = _sc_combine(slot_of_assign, rw.reshape(-1), o_out)
    return y.reshape(Bs, Ss, Hd)


# speedup vs baseline: 1.0705x; 1.0027x over previous
import functools

import jax
import jax.numpy as jnp
from jax import lax
from jax.experimental import pallas as pl
from jax.experimental.pallas import tpu as pltpu
from jax.experimental.pallas import tpu_sc as plsc

E = 64
TOPK = 2
H = 768
F = 1024
N = 4096
A = N * TOPK
T = 128
P = A + E * T
NT = P // T

NC = 2
NS = 16
NW = NC * NS

_SQRT_HALF = 0.7071067811865476


def _gelu_exact(v):
    return 0.5 * v * (1.0 + lax.erf(v * _SQRT_HALF))


_RB = 512


def _router_body(x_ref, gw_ref, gb_ref, idx_ref, rw_ref, rank_ref, g_ref,
                 xp_ref, cnt_ref):
    blk = pl.program_id(0)
    x = x_ref[...]
    xi = lax.bitcast_convert_type(x, jnp.int32) + jnp.int32(0x8000)
    xlo = jnp.right_shift(xi[:, :H // 2], 16) & jnp.int32(0xFFFF)
    xp_ref[...] = (xi[:, H // 2:] & jnp.int32(-65536)) | xlo
    logits = lax.dot_general(x, gw_ref[...], (((1,), (1,)), ((), ())),
                             preferred_element_type=jnp.float32)
    logits = logits + gb_ref[...][None, :]
    ids = lax.broadcasted_iota(jnp.int32, logits.shape, 1)
    neg = jnp.float32(jnp.finfo(jnp.float32).min)
    m1 = jnp.max(logits, axis=1, keepdims=True)
    a1 = jnp.min(jnp.where(logits == m1, ids, E), axis=1, keepdims=True)
    l2 = jnp.where(ids == a1, neg, logits)
    m2 = jnp.max(l2, axis=1, keepdims=True)
    a2 = jnp.min(jnp.where(l2 == m2, ids, E), axis=1, keepdims=True)
    t = jnp.exp(m2 - m1)
    w1 = 1.0 / (1.0 + t)
    idx_ref[...] = jnp.concatenate([a1, a2], axis=1)
    rw_ref[...] = jnp.concatenate([w1, t * w1], axis=1)

    @pl.when(blk == 0)
    def _():
        cnt_ref[...] = jnp.zeros((1, E), jnp.float32)

    oh0 = (ids == a1).astype(jnp.float32)
    oh1 = (ids == a2).astype(jnp.float32)
    s = oh0 + oh1
    incl = s
    k = 1
    while k < _RB:
        shifted = jnp.concatenate(
            [jnp.zeros((k, E), jnp.float32), incl[:-k, :]], axis=0)
        incl = incl + shifted
        k *= 2
    excl = incl - s
    base = cnt_ref[...] + excl
    r0 = jnp.sum(oh0 * base, axis=1, keepdims=True)
    r1 = jnp.sum(oh1 * base, axis=1, keepdims=True)
    rank_ref[...] = jnp.concatenate([r0, r1], axis=1).astype(jnp.int32)
    cnt_new = cnt_ref[...] + incl[-1:, :]
    cnt_ref[...] = cnt_new
    g_ref[...] = cnt_new.astype(jnp.int32)


def _router(xf, gate_w, gate_b):
    return pl.pallas_call(
        _router_body,
        grid=(N // _RB,),
        in_specs=[
            pl.BlockSpec((_RB, H), lambda i: (i, 0)),
            pl.BlockSpec((E, H), lambda i: (0, 0)),
            pl.BlockSpec((E,), lambda i: (0,)),
        ],
        out_specs=[
            pl.BlockSpec((_RB, TOPK), lambda i: (i, 0)),
            pl.BlockSpec((_RB, TOPK), lambda i: (i, 0)),
            pl.BlockSpec((_RB, TOPK), lambda i: (i, 0)),
            pl.BlockSpec((1, E), lambda i: (0, 0)),
            pl.BlockSpec((_RB, H // 2), lambda i: (i, 0)),
        ],
        out_shape=[
            jax.ShapeDtypeStruct((N, TOPK), jnp.int32),
            jax.ShapeDtypeStruct((N, TOPK), jnp.float32),
            jax.ShapeDtypeStruct((N, TOPK), jnp.int32),
            jax.ShapeDtypeStruct((1, E), jnp.int32),
            jax.ShapeDtypeStruct((N, H // 2), jnp.int32),
        ],
        scratch_shapes=[pltpu.VMEM((1, E), jnp.float32)],
        compiler_params=pltpu.CompilerParams(dimension_semantics=("arbitrary",)),
    )(xf, gate_w, gate_b)


_DTW = N // NW
_DHF = _DTW // 2


@functools.lru_cache(maxsize=None)
def _make_sc_dispatch():
    mesh = plsc.VectorSubcoreMesh(
        core_axis_name="c", subcore_axis_name="s",
        num_cores=NC, num_subcores=NS)

    @functools.partial(
        pl.kernel,
        out_type=jax.ShapeDtypeStruct((P, H // 2), jnp.int32),
        mesh=mesh,
        scratch_types=[
            pltpu.VMEM((_DTW, H // 2), jnp.int32),
            [pltpu.VMEM((_DHF,), jnp.int32) for _ in range(4)],
            pltpu.SemaphoreType.DMA,
        ],
    )
    def dispatch_k(sk0_hbm, sk1_hbm, x_hbm, out_hbm, buf, idxs, sem):
        wid = lax.axis_index("s") * NC + lax.axis_index("c")
        t0 = wid * _DTW
        pltpu.sync_copy(x_hbm.at[pl.ds(t0, _DTW)], buf)
        pltpu.sync_copy(sk0_hbm.at[pl.ds(t0, _DHF)], idxs[0])
        pltpu.sync_copy(sk0_hbm.at[pl.ds(t0 + _DHF, _DHF)], idxs[1])
        pltpu.sync_copy(sk1_hbm.at[pl.ds(t0, _DHF)], idxs[2])
        pltpu.sync_copy(sk1_hbm.at[pl.ds(t0 + _DHF, _DHF)], idxs[3])
        handles = [
            pltpu.async_copy(buf.at[pl.ds(0, _DHF)], out_hbm.at[idxs[0]], sem),
            pltpu.async_copy(buf.at[pl.ds(_DHF, _DHF)], out_hbm.at[idxs[1]], sem),
            pltpu.async_copy(buf.at[pl.ds(0, _DHF)], out_hbm.at[idxs[2]], sem),
            pltpu.async_copy(buf.at[pl.ds(_DHF, _DHF)], out_hbm.at[idxs[3]], sem),
        ]
        for h in handles:
            h.wait()

    return dispatch_k


def _sc_dispatch(slots_k0, slots_k1, xf):
    return _make_sc_dispatch()(slots_k0, slots_k1, xf)


def _ffn_body(er_ref, nr_ref, x_ref, w1_ref, b1_ref, w2_ref, b2_ref, o_ref):
    t = pl.program_id(0)

    @pl.when(t < nr_ref[0])
    def _():
        xi = x_ref[...]
        x = jnp.concatenate(
            [lax.bitcast_convert_type(jnp.left_shift(xi, 16), jnp.float32),
             lax.bitcast_convert_type(xi & jnp.int32(-65536), jnp.float32)],
            axis=1)
        h = lax.dot_general(x, w1_ref[0], (((1,), (1,)), ((), ())),
                            preferred_element_type=jnp.float32)
        h = _gelu_exact(h + b1_ref[0])
        o = lax.dot_general(h, w2_ref[0], (((1,), (1,)), ((), ())),
                            preferred_element_type=jnp.float32)
        o = o + b2_ref[0]
        oi = lax.bitcast_convert_type(o, jnp.int32) + jnp.int32(0x8000)
        lo = jnp.right_shift(oi[:, :H // 2], 16) & jnp.int32(0xFFFF)
        hi = oi[:, H // 2:] & jnp.int32(-65536)
        o_ref[...] = hi | lo


def _gmm(expert_of_tile, n_active, x_sorted, fc1_w, fc1_b, fc2_w, fc2_b):
    def rowblk(t, er, nr):
        return (jnp.minimum(t, nr[0] - 1), 0)

    grid_spec = pltpu.PrefetchScalarGridSpec(
        num_scalar_prefetch=2,
        grid=(NT,),
        in_specs=[
            pl.BlockSpec((T, H // 2), rowblk),
            pl.BlockSpec((1, F, H), lambda t, er, nr: (er[t], 0, 0)),
            pl.BlockSpec((1, 1, F), lambda t, er, nr: (er[t], 0, 0)),
            pl.BlockSpec((1, H, F), lambda t, er, nr: (er[t], 0, 0)),
            pl.BlockSpec((1, 1, H), lambda t, er, nr: (er[t], 0, 0)),
        ],
        out_specs=pl.BlockSpec((T, H // 2), rowblk),
    )
    return pl.pallas_call(
        _ffn_body,
        grid_spec=grid_spec,
        out_shape=jax.ShapeDtypeStruct((P, H // 2), jnp.int32),
        compiler_params=pltpu.CompilerParams(dimension_semantics=("arbitrary",)),
    )(expert_of_tile, n_active, x_sorted, fc1_w, fc1_b.reshape(E, 1, F),
      fc2_w, fc2_b.reshape(E, 1, H))


_CT = 8
_CDEPTH = 4


@functools.lru_cache(maxsize=None)
def _make_sc_combine():
    mesh = plsc.VectorSubcoreMesh(
        core_axis_name="c", subcore_axis_name="s",
        num_cores=NC, num_subcores=NS)
    n_tok = N // NW
    nch = n_tok // _CT

    @functools.partial(
        pl.kernel,
        out_type=jax.ShapeDtypeStruct((N, H), jnp.float32),
        mesh=mesh,
        scratch_types=[
            pltpu.VMEM((TOPK * n_tok,), jnp.int32),
            [pltpu.VMEM((2 * _CT, H // 2), jnp.int32) for _ in range(_CDEPTH)],
            [pltpu.VMEM((_CT, H), jnp.float32) for _ in range(_CDEPTH)],
            pltpu.VMEM((TOPK * n_tok,), jnp.float32),
            [pltpu.SemaphoreType.DMA for _ in range(_CDEPTH)],
            [pltpu.SemaphoreType.DMA for _ in range(_CDEPTH)],
        ],
        compiler_params=pltpu.CompilerParams(needs_layout_passes=False),
    )
    def combine_k(slots_hbm, rw_hbm, o_hbm, y_hbm, idx_all, rows_v, y_v, w_v,
                  gsem, wsem):
        wid = lax.axis_index("s") * NC + lax.axis_index("c")
        base_t = wid * n_tok
        pltpu.sync_copy(slots_hbm.at[pl.ds(TOPK * base_t, TOPK * n_tok)],
                        idx_all)
        pltpu.sync_copy(rw_hbm.at[pl.ds(TOPK * base_t, TOPK * n_tok)], w_v)

        def start_gather(j, b):
            idx = idx_all.at[pl.ds(TOPK * _CT * j, TOPK * _CT)]
            return pltpu.async_copy(o_hbm.at[idx], rows_v[b], gsem[b])

        gh = [start_gather(j, j) for j in range(_CDEPTH)]
        wh = [None] * _CDEPTH
        for j in range(nch):
            b = j % _CDEPTH
            gh[b].wait()
            if wh[b] is not None:
                wh[b].wait()

            def per_tok(i, c1, j=j, b=b):
                iw = TOPK * (j * _CT + i)
                wa = plsc.load_gather(w_v, [jnp.full((16,), iw, jnp.int32)])
                wb = plsc.load_gather(w_v, [jnp.full((16,), iw + 1,
                                                     jnp.int32)])
                hmask = jnp.full((16,), -65536, jnp.int32)
                for c in range(H // 32):
                    aw = rows_v[b][2 * i, pl.ds(c * 16, 16)]
                    bw = rows_v[b][2 * i + 1, pl.ds(c * 16, 16)]
                    a_lo = plsc.bitcast(jnp.left_shift(aw, 16), jnp.float32)
                    b_lo = plsc.bitcast(jnp.left_shift(bw, 16), jnp.float32)
                    a_hi = plsc.bitcast(aw & hmask, jnp.float32)
                    b_hi = plsc.bitcast(bw & hmask, jnp.float32)
                    y_v[b][i, pl.ds(c * 16, 16)] = wa * a_lo + wb * b_lo
                    y_v[b][i, pl.ds(H // 2 + c * 16, 16)] = (
                        wa * a_hi + wb * b_hi)
                return c1

            lax.fori_loop(0, _CT, per_tok, 0)
            wh[b] = pltpu.async_copy(
                y_v[b], y_hbm.at[pl.ds(base_t + j * _CT, _CT)], wsem[b])
            if j + _CDEPTH < nch:
                gh[b] = start_gather(j + _CDEPTH, b)
        for h in wh:
            if h is not None:
                h.wait()

    return combine_k


def _sc_combine(slot_of_assign, rw_flat, o_out):
    return _make_sc_combine()(slot_of_assign, rw_flat, o_out)


def _dispatch_indices(top_idx, rank, g_row):
    g = g_row[0]
    gpad = ((g + T - 1) // T) * T
    padded_end = jnp.cumsum(gpad).astype(jnp.int32)
    padded_start = padded_end - gpad
    slot_of_assign = (padded_start[top_idx] + rank).reshape(-1)
    tile_rows = jnp.arange(NT, dtype=jnp.int32) * T
    expert_of_tile = jnp.minimum(
        jnp.searchsorted(padded_end, tile_rows, side="right"), E - 1
    ).astype(jnp.int32)
    n_active = (padded_end[-1] // T).astype(jnp.int32).reshape(1)
    return slot_of_assign, expert_of_tile, n_active


def kernel(x, gate_w, gate_b, fc1_w, fc1_b, fc2_w, fc2_b):
    Bs, Ss, Hd = x.shape
    xf = x.reshape(-1, Hd)
    top_idx, rw, rank, g_row, xp = _router(xf, gate_w, gate_b)
    slot_of_assign, expert_of_tile, n_active = _dispatch_indices(
        top_idx, rank, g_row)
    slot_tk = slot_of_assign.reshape(N, TOPK)
    x_sorted = _sc_dispatch(slot_tk[:, 0], slot_tk[:, 1], xp)
    o_out = _gmm(expert_of_tile, n_active, x_sorted, fc1_w, fc1_b,
                 fc2_w, fc2_b)
    y = _sc_combine(slot_of_assign, rw.reshape(-1), o_out)
    return y.reshape(Bs, Ss, Hd)
